# Initial kernel scaffold; baseline (speedup 1.0000x reference)
import jax
import jax.numpy as jnp
from jax.experimental import pallas as pl


def kernel(x, attention_scores, bias_table):
    raise NotImplementedError("write your pallas kernel here")


# speedup vs baseline: 61.0045x; 61.0045x over previous
import functools

import jax
import jax.numpy as jnp
from jax import lax
from jax.experimental import pallas as pl
from jax.experimental.pallas import tpu as pltpu
from jax.experimental.pallas import tpu_sc as plsc

NUM_BUCKETS = 32
NUM_HEADS = 16
SEQ = 2048
WIDTH = 2 * SEQ
NSKEW = 8
THRESH = (12, 16, 23, 32, 46, 64, 91)
NC, NS, LANES = 2, 16, 16


def _bucket_of(d):
    ad = jnp.abs(d)
    zero = jnp.zeros_like(d)
    one = jnp.ones_like(d)
    large = jnp.full_like(d, 8)
    for t in THRESH:
        large = large + jnp.minimum(jnp.maximum(ad - (t - 1), zero), one)
    b = jnp.minimum(ad, large)
    return b + 16 * jnp.minimum(jnp.maximum(d, zero), one)


def _build_wskew_sc(bias_table):
    rows = NUM_HEADS * NSKEW
    nworkers = NC * NS
    rows_per_worker = rows // nworkers
    mesh = plsc.VectorSubcoreMesh(
        core_axis_name="c", subcore_axis_name="s", num_cores=NC, num_subcores=NS
    )

    @functools.partial(
        pl.kernel,
        mesh=mesh,
        out_type=jax.ShapeDtypeStruct((NUM_HEADS, NSKEW, WIDTH), jnp.float32),
        scratch_types=[
            pltpu.VMEM((NUM_BUCKETS * NUM_HEADS,), jnp.float32),
            pltpu.VMEM((WIDTH,), jnp.float32),
        ],
        compiler_params=pltpu.CompilerParams(needs_layout_passes=False),
    )
    def sc_kernel(table_hbm, out_hbm, table_v, row_v):
        wid = lax.axis_index("s") * NC + lax.axis_index("c")
        pltpu.sync_copy(table_hbm, table_v)
        for j in range(rows_per_worker):
            r = wid * rows_per_worker + j
            h = r // NSKEW
            i = r % NSKEW

            def chunk(c, carry):
                t = lax.iota(jnp.int32, LANES) + c * LANES
                d = t - i - SEQ
                idx = _bucket_of(d) * NUM_HEADS + h
                row_v[pl.ds(c * LANES, LANES)] = plsc.load_gather(table_v, [idx])
                return carry

            lax.fori_loop(0, WIDTH // LANES, chunk, 0)
            pltpu.sync_copy(row_v, out_hbm.at[h, i])

    return sc_kernel(bias_table.reshape(-1))


def _add_bias_tc(scores, wskew, block_q=256):
    _, heads, seq_q, seq_k = scores.shape
    grid = (heads, seq_q // block_q)

    def body(s_ref, w_ref, o_ref):
        qb = pl.program_id(1)
        w = w_ref[0]
        for g in range(block_q // NSKEW):
            start = qb * block_q + g * NSKEW
            shift = WIDTH - (SEQ - start)
            rolled = pltpu.roll(w, shift, axis=1)
            sl = slice(g * NSKEW, (g + 1) * NSKEW)
            o_ref[0, 0, sl, :] = s_ref[0, 0, sl, :] + rolled[:, :seq_k]

    return pl.pallas_call(
        body,
        grid=grid,
        in_specs=[
            pl.BlockSpec((1, 1, block_q, seq_k), lambda h, q: (0, h, q, 0)),
            pl.BlockSpec((1, NSKEW, WIDTH), lambda h, q: (h, 0, 0)),
        ],
        out_specs=pl.BlockSpec((1, 1, block_q, seq_k), lambda h, q: (0, h, q, 0)),
        out_shape=jax.ShapeDtypeStruct(scores.shape, scores.dtype),
    )(scores, wskew)


@jax.jit
def kernel(x, attention_scores, bias_table):
    del x
    wskew = _build_wskew_sc(bias_table)
    return _add_bias_tc(attention_scores, wskew)
